# baseline (speedup 1.0000x reference)
import math
from functools import partial

import jax
import jax.numpy as jnp
from jax.experimental import pallas as pl

N, D, H = 2048, 2048, 16
DH = D // H
SCALE = 1.0 / math.sqrt(DH)
EPS = 1e-5

TM = 256
TQ = 256

_DIMS = (((1,), (1,)), ((), ()))


def _proj_body(x_ref, w_ref, b_ref, o_ref):
    acc = jax.lax.dot_general(
        x_ref[...], w_ref[...], _DIMS, preferred_element_type=jnp.float32)
    o_ref[...] = (acc + b_ref[...]).astype(o_ref.dtype)


def _proj(x, W, b, out_dtype=jnp.float32):
    return pl.pallas_call(
        _proj_body,
        grid=(N // TM,),
        in_specs=[
            pl.BlockSpec((TM, D), lambda i: (i, 0)),
            pl.BlockSpec((D, D), lambda i: (0, 0)),
            pl.BlockSpec((1, D), lambda i: (0, 0)),
        ],
        out_specs=pl.BlockSpec((TM, D), lambda i: (i, 0)),
        out_shape=jax.ShapeDtypeStruct((N, D), out_dtype),
    )(x, W, b.reshape(1, D))


def _poincare_bf16(x):
    x = x.astype(jnp.float32)
    norm = jnp.sqrt(jnp.sum(x * x, axis=-1, keepdims=True))
    max_norm = 1.0 - EPS
    scale = jnp.where(norm > max_norm, max_norm / jnp.maximum(norm, 1e-12), 1.0)
    return (x * scale).astype(jnp.bfloat16)


def _attn_body(q_ref, k_ref, v_ref, o_ref):
    q = _poincare_bf16(q_ref[...])
    k = _poincare_bf16(k_ref[...])
    s = jax.lax.dot_general(
        q, k, _DIMS, preferred_element_type=jnp.float32)
    s = s * SCALE
    m = jnp.max(s, axis=-1, keepdims=True)
    p = jnp.exp(s - m)
    l = jnp.sum(p, axis=-1, keepdims=True)
    o = jnp.dot(p.astype(jnp.bfloat16), v_ref[...],
                preferred_element_type=jnp.float32)
    o_ref[...] = (o / l).astype(jnp.bfloat16)


def _attention(q, k, v):
    return pl.pallas_call(
        _attn_body,
        grid=(H, N // TQ),
        in_specs=[
            pl.BlockSpec((TQ, DH), lambda h, i: (i, h)),
            pl.BlockSpec((N, DH), lambda h, i: (0, h)),
            pl.BlockSpec((N, DH), lambda h, i: (0, h)),
        ],
        out_specs=pl.BlockSpec((TQ, DH), lambda h, i: (i, h)),
        out_shape=jax.ShapeDtypeStruct((N, D), jnp.bfloat16),
    )(q, k, v)


def kernel(query, key_, value, Wq, bq, Wk, bk, Wv, bv, Wo, bo):
    bf = jnp.bfloat16
    x_q = query.reshape(N, D).astype(bf)
    x_k = key_.reshape(N, D).astype(bf)
    x_v = value.reshape(N, D).astype(bf)
    q = _proj(x_q, Wq.astype(bf), bq)
    k = _proj(x_k, Wk.astype(bf), bk)
    v = _proj(x_v, Wv.astype(bf), bv, out_dtype=bf)
    o = _attention(q, k, v)
    out = _proj(o, Wo.astype(bf), bo)
    return out.reshape(1, N, D)


# speedup vs baseline: 1.1294x; 1.1294x over previous
import math

import jax
import jax.numpy as jnp
from jax.experimental import pallas as pl

N, D, H = 2048, 2048, 16
DH = D // H
SCALE = 1.0 / math.sqrt(DH)
EPS = 1e-5

TM = 256
TQ = 256

_DIMS = (((1,), (1,)), ((), ()))


def _proj_body(x_ref, w_ref, b_ref, o_ref):
    acc = jax.lax.dot_general(
        x_ref[...], w_ref[...], _DIMS, preferred_element_type=jnp.float32)
    o_ref[...] = acc + b_ref[...]


def _proj(x, W, b):
    return pl.pallas_call(
        _proj_body,
        grid=(N // TM,),
        in_specs=[
            pl.BlockSpec((TM, D), lambda i: (i, 0)),
            pl.BlockSpec((D, D), lambda i: (0, 0)),
            pl.BlockSpec((1, D), lambda i: (0, 0)),
        ],
        out_specs=pl.BlockSpec((TM, D), lambda i: (i, 0)),
        out_shape=jax.ShapeDtypeStruct((N, D), jnp.float32),
    )(x, W, b.reshape(1, D))


def _poincare(x):
    norm = jnp.sqrt(jnp.sum(x * x, axis=-1, keepdims=True))
    max_norm = 1.0 - EPS
    scale = jnp.where(norm > max_norm, max_norm / jnp.maximum(norm, 1e-12), 1.0)
    return x * scale


def _attn_body(q_ref, k_ref, v_ref, o_ref):
    q = _poincare(q_ref[...])
    k = _poincare(k_ref[...])
    s = jax.lax.dot_general(
        q, k, _DIMS, preferred_element_type=jnp.float32)
    s = s * SCALE
    m = jnp.max(s, axis=-1, keepdims=True)
    p = jnp.exp(s - m)
    l = jnp.sum(p, axis=-1, keepdims=True)
    o = jnp.dot(p, v_ref[...], preferred_element_type=jnp.float32)
    o_ref[...] = o / l


def _attention(q, k, v):
    return pl.pallas_call(
        _attn_body,
        grid=(H, N // TQ),
        in_specs=[
            pl.BlockSpec((TQ, DH), lambda h, i: (i, h)),
            pl.BlockSpec((N, DH), lambda h, i: (0, h)),
            pl.BlockSpec((N, DH), lambda h, i: (0, h)),
        ],
        out_specs=pl.BlockSpec((TQ, DH), lambda h, i: (i, h)),
        out_shape=jax.ShapeDtypeStruct((N, D), jnp.float32),
    )(q, k, v)


def kernel(query, key_, value, Wq, bq, Wk, bk, Wv, bv, Wo, bo):
    x_q = query.reshape(N, D)
    x_k = key_.reshape(N, D)
    x_v = value.reshape(N, D)
    q = _proj(x_q, Wq, bq)
    k = _proj(x_k, Wk, bk)
    v = _proj(x_v, Wv, bv)
    o = _attention(q, k, v)
    out = _proj(o, Wo, bo)
    return out.reshape(1, N, D)


# speedup vs baseline: 1.4893x; 1.3186x over previous
import math
from functools import partial

import jax
import jax.numpy as jnp
from jax.experimental import pallas as pl

N, D, H = 2048, 2048, 16
DH = D // H
SCALE = 1.0 / math.sqrt(DH)
EPS = 1e-5

TM = 256
TQ = 256

_DIMS_NT = (((1,), (1,)), ((), ()))
_DIMS_NN = (((1,), (0,)), ((), ()))


def _proj_body(x_ref, w_ref, b_ref, g_ref, gt_ref, o_ref, *, poincare):
    y = jax.lax.dot_general(
        x_ref[...], w_ref[...], _DIMS_NT, preferred_element_type=jnp.float32)
    y = y + b_ref[...]
    if poincare:
        gs = jax.lax.dot_general(
            y * y, g_ref[...], _DIMS_NN, preferred_element_type=jnp.float32)
        norm = jnp.sqrt(gs)
        max_norm = 1.0 - EPS
        scale = jnp.where(norm > max_norm,
                          max_norm / jnp.maximum(norm, 1e-12), 1.0)
        y = y * jax.lax.dot_general(
            scale, gt_ref[...], _DIMS_NN, preferred_element_type=jnp.float32)
    o_ref[...] = y


def _proj(x, W, b, g, gt, poincare):
    return pl.pallas_call(
        partial(_proj_body, poincare=poincare),
        grid=(N // TM,),
        in_specs=[
            pl.BlockSpec((TM, D), lambda i: (i, 0)),
            pl.BlockSpec((D, D), lambda i: (0, 0)),
            pl.BlockSpec((1, D), lambda i: (0, 0)),
            pl.BlockSpec((D, H), lambda i: (0, 0)),
            pl.BlockSpec((H, D), lambda i: (0, 0)),
        ],
        out_specs=pl.BlockSpec((TM, D), lambda i: (i, 0)),
        out_shape=jax.ShapeDtypeStruct((N, D), jnp.float32),
    )(x, W, b.reshape(1, D), g, gt)


def _attn_body(q_ref, k_ref, v_ref, o_ref):
    q = q_ref[...] * SCALE
    s = jax.lax.dot_general(
        q, k_ref[...], _DIMS_NT, preferred_element_type=jnp.float32)
    p = jnp.exp(s)
    l = jnp.sum(p, axis=-1, keepdims=True)
    o = jnp.dot(p, v_ref[...], preferred_element_type=jnp.float32)
    o_ref[...] = o / l


def _attention(q, k, v):
    return pl.pallas_call(
        _attn_body,
        grid=(H, N // TQ),
        in_specs=[
            pl.BlockSpec((TQ, DH), lambda h, i: (i, h)),
            pl.BlockSpec((N, DH), lambda h, i: (0, h)),
            pl.BlockSpec((N, DH), lambda h, i: (0, h)),
        ],
        out_specs=pl.BlockSpec((TQ, DH), lambda h, i: (i, h)),
        out_shape=jax.ShapeDtypeStruct((N, D), jnp.float32),
    )(q, k, v)


def kernel(query, key_, value, Wq, bq, Wk, bk, Wv, bv, Wo, bo):
    g = (jnp.arange(D)[:, None] // DH == jnp.arange(H)[None, :]).astype(jnp.float32)
    gt = g.T
    x_q = query.reshape(N, D)
    x_k = key_.reshape(N, D)
    x_v = value.reshape(N, D)
    q = _proj(x_q, Wq, bq, g, gt, poincare=True)
    k = _proj(x_k, Wk, bk, g, gt, poincare=True)
    v = _proj(x_v, Wv, bv, g, gt, poincare=False)
    o = _attention(q, k, v)
    out = _proj(o, Wo, bo, g, gt, poincare=False)
    return out.reshape(1, N, D)


# speedup vs baseline: 1.5146x; 1.0170x over previous
import math
from functools import partial

import jax
import jax.numpy as jnp
from jax.experimental import pallas as pl

N, D, H = 2048, 2048, 16
DH = D // H
SCALE = 1.0 / math.sqrt(DH)
EPS = 1e-5

TM = 256
TQ = 256

_DIMS_NT = (((1,), (1,)), ((), ()))
_DIMS_NN = (((1,), (0,)), ((), ()))


def _proj_body(x_ref, w_ref, b_ref, g_ref, gt_ref, o_ref, *, poincare):
    y = jax.lax.dot_general(
        x_ref[...].astype(jnp.bfloat16), w_ref[...].astype(jnp.bfloat16),
        _DIMS_NT, preferred_element_type=jnp.float32)
    y = y + b_ref[...]
    if poincare:
        gs = jax.lax.dot_general(
            y * y, g_ref[...], _DIMS_NN, preferred_element_type=jnp.float32)
        norm = jnp.sqrt(gs)
        max_norm = 1.0 - EPS
        scale = jnp.where(norm > max_norm,
                          max_norm / jnp.maximum(norm, 1e-12), 1.0)
        y = y * jax.lax.dot_general(
            scale, gt_ref[...], _DIMS_NN, preferred_element_type=jnp.float32)
    o_ref[...] = y


def _proj(x, W, b, g, gt, poincare):
    return pl.pallas_call(
        partial(_proj_body, poincare=poincare),
        grid=(N // TM,),
        in_specs=[
            pl.BlockSpec((TM, D), lambda i: (i, 0)),
            pl.BlockSpec((D, D), lambda i: (0, 0)),
            pl.BlockSpec((1, D), lambda i: (0, 0)),
            pl.BlockSpec((D, H), lambda i: (0, 0)),
            pl.BlockSpec((H, D), lambda i: (0, 0)),
        ],
        out_specs=pl.BlockSpec((TM, D), lambda i: (i, 0)),
        out_shape=jax.ShapeDtypeStruct((N, D), jnp.float32),
    )(x, W, b.reshape(1, D), g, gt)


def _attn_body(q_ref, k_ref, v_ref, o_ref):
    q = (q_ref[...] * SCALE).astype(jnp.bfloat16)
    s = jax.lax.dot_general(
        q, k_ref[...].astype(jnp.bfloat16), _DIMS_NT,
        preferred_element_type=jnp.float32)
    p = jnp.exp(s)
    l = jnp.sum(p, axis=-1, keepdims=True)
    o = jnp.dot(p.astype(jnp.bfloat16), v_ref[...].astype(jnp.bfloat16),
                preferred_element_type=jnp.float32)
    o_ref[...] = o / l


def _attention(q, k, v):
    return pl.pallas_call(
        _attn_body,
        grid=(H, N // TQ),
        in_specs=[
            pl.BlockSpec((TQ, DH), lambda h, i: (i, h)),
            pl.BlockSpec((N, DH), lambda h, i: (0, h)),
            pl.BlockSpec((N, DH), lambda h, i: (0, h)),
        ],
        out_specs=pl.BlockSpec((TQ, DH), lambda h, i: (i, h)),
        out_shape=jax.ShapeDtypeStruct((N, D), jnp.float32),
    )(q, k, v)


def kernel(query, key_, value, Wq, bq, Wk, bk, Wv, bv, Wo, bo):
    g = (jnp.arange(D)[:, None] // DH == jnp.arange(H)[None, :]).astype(jnp.float32)
    gt = g.T
    x_q = query.reshape(N, D)
    x_k = key_.reshape(N, D)
    x_v = value.reshape(N, D)
    q = _proj(x_q, Wq, bq, g, gt, poincare=True)
    k = _proj(x_k, Wk, bk, g, gt, poincare=True)
    v = _proj(x_v, Wv, bv, g, gt, poincare=False)
    o = _attention(q, k, v)
    out = _proj(o, Wo, bo, g, gt, poincare=False)
    return out.reshape(1, N, D)


# speedup vs baseline: 1.5837x; 1.0456x over previous
import math
from functools import partial

import jax
import jax.numpy as jnp
from jax.experimental import pallas as pl

N, D, H = 2048, 2048, 16
DH = D // H
SCALE = 1.0 / math.sqrt(DH)
EPS = 1e-5

TM = 256
TQ = 256

_DIMS_NT = (((1,), (1,)), ((), ()))
_DIMS_NN = (((1,), (0,)), ((), ()))

_BF = jnp.bfloat16


def _proj_body(x_ref, w_ref, b_ref, g_ref, gt_ref, o_ref, *, poincare):
    y = jax.lax.dot_general(
        x_ref[...].astype(_BF), w_ref[...].astype(_BF),
        _DIMS_NT, preferred_element_type=jnp.float32)
    y = y + b_ref[...]
    if poincare:
        gs = jax.lax.dot_general(
            y * y, g_ref[...], _DIMS_NN, preferred_element_type=jnp.float32)
        norm = jnp.sqrt(gs)
        max_norm = 1.0 - EPS
        scale = jnp.where(norm > max_norm,
                          max_norm / jnp.maximum(norm, 1e-12), 1.0)
        y = y * jax.lax.dot_general(
            scale, gt_ref[...], _DIMS_NN, preferred_element_type=jnp.float32)
    o_ref[...] = y.astype(o_ref.dtype)


def _proj(x, W, b, g, gt, poincare, out_dtype):
    return pl.pallas_call(
        partial(_proj_body, poincare=poincare),
        grid=(N // TM,),
        in_specs=[
            pl.BlockSpec((TM, D), lambda i: (i, 0)),
            pl.BlockSpec((D, D), lambda i: (0, 0)),
            pl.BlockSpec((1, D), lambda i: (0, 0)),
            pl.BlockSpec((D, H), lambda i: (0, 0)),
            pl.BlockSpec((H, D), lambda i: (0, 0)),
        ],
        out_specs=pl.BlockSpec((TM, D), lambda i: (i, 0)),
        out_shape=jax.ShapeDtypeStruct((N, D), out_dtype),
    )(x, W, b.reshape(1, D), g, gt)


def _attn_body(q_ref, k_ref, v_ref, o_ref):
    q = (q_ref[...].astype(jnp.float32) * SCALE).astype(_BF)
    s = jax.lax.dot_general(
        q, k_ref[...], _DIMS_NT, preferred_element_type=jnp.float32)
    p = jnp.exp(s)
    l = jnp.sum(p, axis=-1, keepdims=True)
    o = jnp.dot(p.astype(_BF), v_ref[...], preferred_element_type=jnp.float32)
    o_ref[...] = (o / l).astype(o_ref.dtype)


def _attention(q, k, v):
    return pl.pallas_call(
        _attn_body,
        grid=(H, N // TQ),
        in_specs=[
            pl.BlockSpec((TQ, DH), lambda h, i: (i, h)),
            pl.BlockSpec((N, DH), lambda h, i: (0, h)),
            pl.BlockSpec((N, DH), lambda h, i: (0, h)),
        ],
        out_specs=pl.BlockSpec((TQ, DH), lambda h, i: (i, h)),
        out_shape=jax.ShapeDtypeStruct((N, D), _BF),
    )(q, k, v)


def kernel(query, key_, value, Wq, bq, Wk, bk, Wv, bv, Wo, bo):
    g = (jnp.arange(D)[:, None] // DH == jnp.arange(H)[None, :]).astype(jnp.float32)
    gt = g.T
    x_q = query.reshape(N, D)
    x_k = key_.reshape(N, D)
    x_v = value.reshape(N, D)
    q = _proj(x_q, Wq, bq, g, gt, poincare=True, out_dtype=_BF)
    k = _proj(x_k, Wk, bk, g, gt, poincare=True, out_dtype=_BF)
    v = _proj(x_v, Wv, bv, g, gt, poincare=False, out_dtype=_BF)
    o = _attention(q, k, v)
    out = _proj(o, Wo, bo, g, gt, poincare=False, out_dtype=jnp.float32)
    return out.reshape(1, N, D)


# speedup vs baseline: 1.7433x; 1.1007x over previous
import math
from functools import partial

import jax
import jax.numpy as jnp
from jax.experimental import pallas as pl

N, D, H = 2048, 2048, 16
DH = D // H
SCALE = 1.0 / math.sqrt(DH)
EPS = 1e-5

TM = 256
TQ = 512

_DIMS_NT = (((1,), (1,)), ((), ()))
_DIMS_NN = (((1,), (0,)), ((), ()))

_BF = jnp.bfloat16


def _proj_body(x_ref, w_ref, b_ref, g_ref, gt_ref, o_ref, *, poincare):
    y = jax.lax.dot_general(
        x_ref[...].astype(_BF), w_ref[...].astype(_BF),
        _DIMS_NT, preferred_element_type=jnp.float32)
    y = y + b_ref[...]
    if poincare:
        gs = jax.lax.dot_general(
            y * y, g_ref[...], _DIMS_NN, preferred_element_type=jnp.float32)
        norm = jnp.sqrt(gs)
        max_norm = 1.0 - EPS
        scale = jnp.where(norm > max_norm,
                          max_norm / jnp.maximum(norm, 1e-12), 1.0)
        y = y * jax.lax.dot_general(
            scale, gt_ref[...], _DIMS_NN, preferred_element_type=jnp.float32)
    o_ref[...] = y.astype(o_ref.dtype)


def _proj(x, W, b, g, gt, poincare, out_dtype):
    return pl.pallas_call(
        partial(_proj_body, poincare=poincare),
        grid=(N // TM,),
        in_specs=[
            pl.BlockSpec((TM, D), lambda i: (i, 0)),
            pl.BlockSpec((D, D), lambda i: (0, 0)),
            pl.BlockSpec((1, D), lambda i: (0, 0)),
            pl.BlockSpec((D, H), lambda i: (0, 0)),
            pl.BlockSpec((H, D), lambda i: (0, 0)),
        ],
        out_specs=pl.BlockSpec((TM, D), lambda i: (i, 0)),
        out_shape=jax.ShapeDtypeStruct((N, D), out_dtype),
    )(x, W, b.reshape(1, D), g, gt)


def _attn_body(q_ref, k_ref, v_ref, o_ref):
    q = (q_ref[...].astype(jnp.float32) * SCALE).astype(_BF)
    s = jax.lax.dot_general(
        q, k_ref[...], _DIMS_NT, preferred_element_type=jnp.float32)
    p = jnp.exp(s.astype(_BF))
    l = jnp.sum(p.astype(jnp.float32), axis=-1, keepdims=True)
    o = jnp.dot(p, v_ref[...], preferred_element_type=jnp.float32)
    o_ref[...] = (o / l).astype(o_ref.dtype)


def _attention(q, k, v):
    return pl.pallas_call(
        _attn_body,
        grid=(H, N // TQ),
        in_specs=[
            pl.BlockSpec((TQ, DH), lambda h, i: (i, h)),
            pl.BlockSpec((N, DH), lambda h, i: (0, h)),
            pl.BlockSpec((N, DH), lambda h, i: (0, h)),
        ],
        out_specs=pl.BlockSpec((TQ, DH), lambda h, i: (i, h)),
        out_shape=jax.ShapeDtypeStruct((N, D), _BF),
    )(q, k, v)


def kernel(query, key_, value, Wq, bq, Wk, bk, Wv, bv, Wo, bo):
    g = (jnp.arange(D)[:, None] // DH == jnp.arange(H)[None, :]).astype(jnp.float32)
    gt = g.T
    x_q = query.reshape(N, D)
    x_k = key_.reshape(N, D)
    x_v = value.reshape(N, D)
    q = _proj(x_q, Wq, bq, g, gt, poincare=True, out_dtype=_BF)
    k = _proj(x_k, Wk, bk, g, gt, poincare=True, out_dtype=_BF)
    v = _proj(x_v, Wv, bv, g, gt, poincare=False, out_dtype=_BF)
    o = _attention(q, k, v)
    out = _proj(o, Wo, bo, g, gt, poincare=False, out_dtype=jnp.float32)
    return out.reshape(1, N, D)


# speedup vs baseline: 1.7671x; 1.0137x over previous
import math
from functools import partial

import jax
import jax.numpy as jnp
from jax.experimental import pallas as pl

N, D, H = 2048, 2048, 16
DH = D // H
SCALE = 1.0 / math.sqrt(DH)
EPS = 1e-5

TM = 512
TQ = 512

_DIMS_NT = (((1,), (1,)), ((), ()))
_DIMS_NN = (((1,), (0,)), ((), ()))

_BF = jnp.bfloat16


def _proj_body(x_ref, w_ref, b_ref, g_ref, gt_ref, o_ref, *, poincare):
    y = jax.lax.dot_general(
        x_ref[...].astype(_BF), w_ref[...].astype(_BF),
        _DIMS_NT, preferred_element_type=jnp.float32)
    y = y + b_ref[...]
    if poincare:
        gs = jax.lax.dot_general(
            y * y, g_ref[...], _DIMS_NN, preferred_element_type=jnp.float32)
        norm = jnp.sqrt(gs)
        max_norm = 1.0 - EPS
        scale = jnp.where(norm > max_norm,
                          max_norm / jnp.maximum(norm, 1e-12), 1.0)
        y = y * jax.lax.dot_general(
            scale, gt_ref[...], _DIMS_NN, preferred_element_type=jnp.float32)
    o_ref[...] = y.astype(o_ref.dtype)


def _proj(x, W, b, g, gt, poincare, out_dtype):
    return pl.pallas_call(
        partial(_proj_body, poincare=poincare),
        grid=(N // TM,),
        in_specs=[
            pl.BlockSpec((TM, D), lambda i: (i, 0)),
            pl.BlockSpec((D, D), lambda i: (0, 0)),
            pl.BlockSpec((1, D), lambda i: (0, 0)),
            pl.BlockSpec((D, H), lambda i: (0, 0)),
            pl.BlockSpec((H, D), lambda i: (0, 0)),
        ],
        out_specs=pl.BlockSpec((TM, D), lambda i: (i, 0)),
        out_shape=jax.ShapeDtypeStruct((N, D), out_dtype),
    )(x, W, b.reshape(1, D), g, gt)


def _attn_body(q_ref, k_ref, v_ref, o_ref):
    q = (q_ref[...].astype(jnp.float32) * SCALE).astype(_BF)
    s = jax.lax.dot_general(
        q, k_ref[...], _DIMS_NT, preferred_element_type=jnp.float32)
    p = jnp.exp(s.astype(_BF))
    l = jnp.sum(p.astype(jnp.float32), axis=-1, keepdims=True)
    o = jnp.dot(p, v_ref[...], preferred_element_type=jnp.float32)
    o_ref[...] = (o / l).astype(o_ref.dtype)


def _attention(q, k, v):
    return pl.pallas_call(
        _attn_body,
        grid=(H, N // TQ),
        in_specs=[
            pl.BlockSpec((TQ, DH), lambda h, i: (i, h)),
            pl.BlockSpec((N, DH), lambda h, i: (0, h)),
            pl.BlockSpec((N, DH), lambda h, i: (0, h)),
        ],
        out_specs=pl.BlockSpec((TQ, DH), lambda h, i: (i, h)),
        out_shape=jax.ShapeDtypeStruct((N, D), _BF),
    )(q, k, v)


def kernel(query, key_, value, Wq, bq, Wk, bk, Wv, bv, Wo, bo):
    g = (jnp.arange(D)[:, None] // DH == jnp.arange(H)[None, :]).astype(jnp.float32)
    gt = g.T
    x_q = query.reshape(N, D)
    x_k = key_.reshape(N, D)
    x_v = value.reshape(N, D)
    q = _proj(x_q, Wq, bq, g, gt, poincare=True, out_dtype=_BF)
    k = _proj(x_k, Wk, bk, g, gt, poincare=True, out_dtype=_BF)
    v = _proj(x_v, Wv, bv, g, gt, poincare=False, out_dtype=_BF)
    o = _attention(q, k, v)
    out = _proj(o, Wo, bo, g, gt, poincare=False, out_dtype=jnp.float32)
    return out.reshape(1, N, D)


# speedup vs baseline: 1.8262x; 1.0335x over previous
import math
from functools import partial

import jax
import jax.numpy as jnp
from jax.experimental import pallas as pl
from jax.experimental.pallas import tpu as pltpu

N, D, H = 2048, 2048, 16
DH = D // H
SCALE = 1.0 / math.sqrt(DH)
EPS = 1e-5

TM = 512
TO = 256

_DIMS_NT = (((1,), (1,)), ((), ()))
_DIMS_NN = (((1,), (0,)), ((), ()))
_BF = jnp.bfloat16


def _proj_body(x_ref, w_ref, b_ref, g_ref, gt_ref, o_ref, *, poincare):
    y = jax.lax.dot_general(
        x_ref[...].astype(_BF), w_ref[...].astype(_BF),
        _DIMS_NT, preferred_element_type=jnp.float32)
    y = y + b_ref[...]
    if poincare:
        gs = jax.lax.dot_general(
            y * y, g_ref[...], _DIMS_NN, preferred_element_type=jnp.float32)
        norm = jnp.sqrt(gs)
        max_norm = 1.0 - EPS
        scale = jnp.where(norm > max_norm,
                          max_norm / jnp.maximum(norm, 1e-12), 1.0)
        y = y * jax.lax.dot_general(
            scale, gt_ref[...], _DIMS_NN, preferred_element_type=jnp.float32)
    o_ref[...] = y.astype(o_ref.dtype)


def _proj(x, W, b, g, gt, poincare):
    return pl.pallas_call(
        partial(_proj_body, poincare=poincare),
        grid=(N // TM,),
        in_specs=[
            pl.BlockSpec((TM, D), lambda i: (i, 0)),
            pl.BlockSpec((D, D), lambda i: (0, 0)),
            pl.BlockSpec((1, D), lambda i: (0, 0)),
            pl.BlockSpec((D, H), lambda i: (0, 0)),
            pl.BlockSpec((H, D), lambda i: (0, 0)),
        ],
        out_specs=pl.BlockSpec((TM, D), lambda i: (i, 0)),
        out_shape=jax.ShapeDtypeStruct((N, D), _BF),
    )(x, W, b.reshape(1, D), g, gt)


def _attn_out_body(q_ref, k_ref, v_ref, wo_ref, bo_ref, o_ref, acc_ref):
    for h in range(H):
        sl = slice(h * DH, (h + 1) * DH)
        qh = (q_ref[:, sl].astype(jnp.float32) * SCALE).astype(_BF)
        s = jax.lax.dot_general(
            qh, k_ref[:, sl], _DIMS_NT, preferred_element_type=jnp.float32)
        p = jnp.exp(s.astype(_BF))
        l = jnp.sum(p.astype(jnp.float32), axis=-1, keepdims=True)
        oh = jnp.dot(p, v_ref[:, sl], preferred_element_type=jnp.float32)
        acc_ref[:, sl] = (oh / l).astype(_BF)
    o = jax.lax.dot_general(
        acc_ref[...], wo_ref[...].astype(_BF), _DIMS_NT,
        preferred_element_type=jnp.float32)
    o_ref[...] = o + bo_ref[...]


def _attn_out(q, k, v, Wo, bo):
    return pl.pallas_call(
        _attn_out_body,
        grid=(N // TO,),
        in_specs=[
            pl.BlockSpec((TO, D), lambda i: (i, 0)),
            pl.BlockSpec((N, D), lambda i: (0, 0)),
            pl.BlockSpec((N, D), lambda i: (0, 0)),
            pl.BlockSpec((D, D), lambda i: (0, 0)),
            pl.BlockSpec((1, D), lambda i: (0, 0)),
        ],
        out_specs=pl.BlockSpec((TO, D), lambda i: (i, 0)),
        out_shape=jax.ShapeDtypeStruct((N, D), jnp.float32),
        scratch_shapes=[pltpu.VMEM((TO, D), _BF)],
    )(q, k, v, Wo, bo.reshape(1, D))


def kernel(query, key_, value, Wq, bq, Wk, bk, Wv, bv, Wo, bo):
    g = (jnp.arange(D)[:, None] // DH == jnp.arange(H)[None, :]).astype(jnp.float32)
    gt = g.T
    q = _proj(query.reshape(N, D), Wq, bq, g, gt, poincare=True)
    k = _proj(key_.reshape(N, D), Wk, bk, g, gt, poincare=True)
    v = _proj(value.reshape(N, D), Wv, bv, g, gt, poincare=False)
    out = _attn_out(q, k, v, Wo, bo)
    return out.reshape(1, N, D)


# speedup vs baseline: 1.8824x; 1.0308x over previous
import math
from functools import partial

import jax
import jax.numpy as jnp
from jax.experimental import pallas as pl
from jax.experimental.pallas import tpu as pltpu

N, D, H = 2048, 2048, 16
DH = D // H
SCALE = 1.0 / math.sqrt(DH)
EPS = 1e-5

TM = 512
TO = 256
DA = 2 * DH

_DIMS_NT = (((1,), (1,)), ((), ()))
_DIMS_NN = (((1,), (0,)), ((), ()))
_BF = jnp.bfloat16


def _proj_body(x_ref, w_ref, b_ref, g_ref, gt_ref, o_ref, *, mode):
    y = jax.lax.dot_general(
        x_ref[...].astype(_BF), w_ref[...].astype(_BF),
        _DIMS_NT, preferred_element_type=jnp.float32)
    y = y + b_ref[...]
    if mode in ("q", "k"):
        gs = jax.lax.dot_general(
            y * y, g_ref[...], _DIMS_NN, preferred_element_type=jnp.float32)
        norm = jnp.sqrt(gs)
        max_norm = 1.0 - EPS
        scale = jnp.where(norm > max_norm,
                          max_norm / jnp.maximum(norm, 1e-12), 1.0)
        if mode == "q":
            scale = scale * SCALE
        y = y * jax.lax.dot_general(
            scale, gt_ref[...], _DIMS_NN, preferred_element_type=jnp.float32)
        o_ref[...] = y.astype(_BF)
    else:
        yb = y.astype(_BF)
        for h in range(H):
            o_ref[:, h * DA:h * DA + DH] = yb[:, h * DH:(h + 1) * DH]
            o_ref[:, h * DA + DH:(h + 1) * DA] = jnp.ones(
                (y.shape[0], DH), dtype=_BF)


def _proj(x, W, b, g, gt, mode):
    out_d = 2 * D if mode == "v" else D
    return pl.pallas_call(
        partial(_proj_body, mode=mode),
        grid=(N // TM,),
        in_specs=[
            pl.BlockSpec((TM, D), lambda i: (i, 0)),
            pl.BlockSpec((D, D), lambda i: (0, 0)),
            pl.BlockSpec((1, D), lambda i: (0, 0)),
            pl.BlockSpec((D, H), lambda i: (0, 0)),
            pl.BlockSpec((H, D), lambda i: (0, 0)),
        ],
        out_specs=pl.BlockSpec((TM, out_d), lambda i: (i, 0)),
        out_shape=jax.ShapeDtypeStruct((N, out_d), _BF),
    )(x, W, b.reshape(1, D), g, gt)


def _attn_out_body(q_ref, k_ref, v_ref, wo_ref, bo_ref, o_ref, acc_ref):
    for h in range(H):
        sl = slice(h * DH, (h + 1) * DH)
        s = jax.lax.dot_general(
            q_ref[:, sl], k_ref[:, sl], _DIMS_NT,
            preferred_element_type=jnp.float32)
        p = jnp.exp(s.astype(_BF))
        ov = jnp.dot(p, v_ref[:, h * DA:(h + 1) * DA],
                     preferred_element_type=jnp.float32)
        oh = ov[:, :DH] / ov[:, DH:DH + 1]
        acc_ref[:, sl] = oh.astype(_BF)
    o = jax.lax.dot_general(
        acc_ref[...], wo_ref[...].astype(_BF), _DIMS_NT,
        preferred_element_type=jnp.float32)
    o_ref[...] = o + bo_ref[...]


def _attn_out(q, k, v, Wo, bo):
    return pl.pallas_call(
        _attn_out_body,
        grid=(N // TO,),
        in_specs=[
            pl.BlockSpec((TO, D), lambda i: (i, 0)),
            pl.BlockSpec((N, D), lambda i: (0, 0)),
            pl.BlockSpec((N, 2 * D), lambda i: (0, 0)),
            pl.BlockSpec((D, D), lambda i: (0, 0)),
            pl.BlockSpec((1, D), lambda i: (0, 0)),
        ],
        out_specs=pl.BlockSpec((TO, D), lambda i: (i, 0)),
        out_shape=jax.ShapeDtypeStruct((N, D), jnp.float32),
        scratch_shapes=[pltpu.VMEM((TO, D), _BF)],
    )(q, k, v, Wo, bo.reshape(1, D))


def kernel(query, key_, value, Wq, bq, Wk, bk, Wv, bv, Wo, bo):
    g = (jnp.arange(D)[:, None] // DH == jnp.arange(H)[None, :]).astype(jnp.float32)
    gt = g.T
    q = _proj(query.reshape(N, D), Wq, bq, g, gt, mode="q")
    k = _proj(key_.reshape(N, D), Wk, bk, g, gt, mode="k")
    v = _proj(value.reshape(N, D), Wv, bv, g, gt, mode="v")
    out = _attn_out(q, k, v, Wo, bo)
    return out.reshape(1, N, D)
